# baseline (speedup 1.0000x reference)
import functools

import jax
import jax.numpy as jnp
from jax import lax
from jax.experimental import pallas as pl
from jax.experimental.pallas import tpu as pltpu
from jax.experimental.pallas import tpu_sc as plsc

NC = 2
NS = 16
NW = NC * NS
CH = 128
NB = 4

_SC_PARAMS = pltpu.CompilerParams(use_tc_tiling_on_sc=False)


def _sc_degree(dst2, zeros8, ones8, R, iters):
    rpt = R // NS
    mesh = plsc.VectorSubcoreMesh(core_axis_name="c", subcore_axis_name="s")
    K = 8
    rounds = iters // K

    @functools.partial(
        pl.kernel,
        out_type=jax.ShapeDtypeStruct((NC, R, 8), jnp.float32),
        mesh=mesh,
        scratch_types=[
            pltpu.VMEM((iters, CH), jnp.int32),
            pltpu.VMEM((CH, 8), jnp.float32),
            pltpu.VMEM((rpt, 8), jnp.float32),
            pltpu.VMEM_SHARED((R, 8), jnp.float32),
            pltpu.SemaphoreType.DMA,
        ],
        compiler_params=_SC_PARAMS,
    )
    def k(dst_hbm, zeros_hbm, ones_hbm, out_hbm, idx_v, ones_v, chunk_v,
          hist, sem):
        cid = lax.axis_index("c")
        sid = lax.axis_index("s")
        wid = sid * NC + cid
        row = pl.ds(sid * rpt, rpt)
        pltpu.sync_copy(zeros_hbm.at[row], chunk_v)
        pltpu.sync_copy(chunk_v, hist.at[row])
        pltpu.sync_copy(dst_hbm.at[pl.ds(wid * iters, iters)], idx_v)
        pltpu.sync_copy(ones_hbm, ones_v)
        plsc.subcore_barrier()

        def body(g, carry):
            for b in range(K):
                pltpu.async_copy(
                    ones_v, hist.at[idx_v.at[g * K + b]], sem, add=True)
            for b in range(K):
                pltpu.make_async_copy(
                    ones_v, hist.at[idx_v.at[0]], sem).wait()
            return carry

        lax.fori_loop(0, rounds, body, 0)
        plsc.subcore_barrier()
        pltpu.sync_copy(hist.at[row], chunk_v)
        pltpu.sync_copy(chunk_v, out_hbm.at[cid, row])

    return k(dst2, zeros8, ones8)


def _sc_edge_pass(src2, dst2, table, zeros, R, W, shift, c0, c1, nb):
    rpt = R // NS
    mesh = plsc.VectorSubcoreMesh(core_axis_name="c", subcore_axis_name="s")
    cmax = max(c0, c1)
    NB = nb

    @functools.partial(
        pl.kernel,
        out_type=jax.ShapeDtypeStruct((NC, R, W), jnp.float32),
        mesh=mesh,
        scratch_types=(
            [pltpu.VMEM((cmax, CH), jnp.int32),
             pltpu.VMEM((cmax, CH), jnp.int32)]
            + [pltpu.VMEM((CH, W), jnp.float32) for _ in range(NB)]
            + [pltpu.VMEM_SHARED((R, W), jnp.float32)]
            + [pltpu.SemaphoreType.DMA for _ in range(2 * NB)]
        ),
        compiler_params=_SC_PARAMS,
    )
    def k(src_hbm, dst_hbm, table_hbm, zeros_hbm, out_hbm,
          idxs_v, idxd_v, *bufs_and_sems):
        rows = bufs_and_sems[:NB]
        acc = bufs_and_sems[NB]
        semg = bufs_and_sems[NB + 1:NB + 1 + NB]
        sems = bufs_and_sems[NB + 1 + NB:]
        cid = lax.axis_index("c")
        sid = lax.axis_index("s")
        row = pl.ds(sid * rpt, rpt)

        scope = jax.named_scope
        chunks = []
        o = 0
        while o < rpt:
            c = min(CH, rpt - o)
            chunks.append((o, c))
            o += c
        with scope("ph_init"):
            pltpu.sync_copy(zeros_hbm, rows[0])
            for (o, c) in chunks:
                pltpu.sync_copy(rows[0].at[pl.ds(0, c)],
                                acc.at[pl.ds(sid * rpt + o, c)])

        @pl.when(cid == 0)
        def _load0():
            pltpu.sync_copy(src_hbm.at[pl.ds(sid * c0, c0)],
                            idxs_v.at[pl.ds(0, c0)])
            pltpu.sync_copy(dst_hbm.at[pl.ds(sid * c0, c0)],
                            idxd_v.at[pl.ds(0, c0)])

        @pl.when(cid == 1)
        def _load1():
            pltpu.sync_copy(src_hbm.at[pl.ds(NS * c0 + sid * c1, c1)],
                            idxs_v.at[pl.ds(0, c1)])
            pltpu.sync_copy(dst_hbm.at[pl.ds(NS * c0 + sid * c1, c1)],
                            idxd_v.at[pl.ds(0, c1)])

        rounds = jnp.where(cid == 0, c0 // NB, c1 // NB)
        if shift:
            def sh(i, carry):
                for j in range(CH // 16):
                    sl = pl.ds(j * 16, 16)
                    idxs_v[i, sl] = idxs_v[i, sl] >> 1
                    idxd_v[i, sl] = idxd_v[i, sl] >> 1
                return carry
            lax.fori_loop(0, rounds * NB, sh, 0)
        plsc.subcore_barrier()

        def body(g, carry):
            for b in range(NB):
                @pl.when(g > 0)
                def _drain():
                    pltpu.make_async_copy(
                        rows[b], acc.at[idxd_v.at[0]], sems[b]).wait()
                pltpu.async_copy(
                    table_hbm.at[idxs_v.at[g * NB + b]], rows[b], semg[b])
            for b in range(NB):
                pltpu.make_async_copy(
                    table_hbm.at[idxs_v.at[0]], rows[b], semg[b]).wait()
                pltpu.async_copy(
                    rows[b], acc.at[idxd_v.at[g * NB + b]], sems[b],
                    add=True)
            return carry

        with scope("ph_loop"):
            lax.fori_loop(0, rounds, body, 0)
            for b in range(NB):
                pltpu.make_async_copy(
                    rows[b], acc.at[idxd_v.at[0]], sems[b]).wait()
            plsc.subcore_barrier()
        scope2 = jax.named_scope("ph_out")
        scope2.__enter__()
        live = {}
        for z, (o, c) in enumerate(chunks):
            sl = pl.ds(sid * rpt + o, c)
            b = z % NB
            if b in live:
                pltpu.make_async_copy(
                    rows[b].at[pl.ds(0, live[b])],
                    out_hbm.at[cid, pl.ds(0, live[b])], semg[b]).wait()
            pltpu.sync_copy(acc.at[sl], rows[b].at[pl.ds(0, c)])
            pltpu.async_copy(rows[b].at[pl.ds(0, c)],
                             out_hbm.at[cid, sl], semg[b])
            live[b] = c
        for b, c in live.items():
            pltpu.make_async_copy(
                rows[b].at[pl.ds(0, c)],
                out_hbm.at[cid, pl.ds(0, c)], semg[b]).wait()
        scope2.__exit__(None, None, None)

    return k(src2, dst2, table, zeros)


def _tc_prep_enc(x_pad, W, p0, p1, B=640):
    R, D = x_pad.shape
    H = W.shape[1]

    def body(x_ref, w_ref, p0_ref, p1_ref, o_ref):
        dinv = lax.rsqrt(p0_ref[...] + p1_ref[...] + 1.0)
        o_ref[...] = jnp.dot(x_ref[...], w_ref[...],
                             preferred_element_type=jnp.float32) * dinv

    return pl.pallas_call(
        body,
        grid=(R // B,),
        in_specs=[
            pl.BlockSpec((B, D), lambda i: (i, 0)),
            pl.BlockSpec((D, H), lambda i: (0, 0)),
            pl.BlockSpec((B, 1), lambda i: (i, 0)),
            pl.BlockSpec((B, 1), lambda i: (i, 0)),
        ],
        out_specs=pl.BlockSpec((B, H), lambda i: (i, 0)),
        out_shape=jax.ShapeDtypeStruct((R, H), jnp.float32),
    )(x_pad, W, p0, p1)


def _tc_post_enc(a0, a1, hs, p0, p1, b, B=640):
    R, H = hs.shape

    def body(a0_ref, a1_ref, hs_ref, p0_ref, p1_ref, b_ref, o_ref):
        dinv = lax.rsqrt(p0_ref[...] + p1_ref[...] + 1.0)
        s = (a0_ref[...] + a1_ref[...] + hs_ref[...]) * dinv + b_ref[...]
        o_ref[...] = jnp.maximum(s, 0.0)

    return pl.pallas_call(
        body,
        grid=(R // B,),
        in_specs=[
            pl.BlockSpec((B, H), lambda i: (i, 0)),
            pl.BlockSpec((B, H), lambda i: (i, 0)),
            pl.BlockSpec((B, H), lambda i: (i, 0)),
            pl.BlockSpec((B, 1), lambda i: (i, 0)),
            pl.BlockSpec((B, 1), lambda i: (i, 0)),
            pl.BlockSpec((1, H), lambda i: (0, 0)),
        ],
        out_specs=pl.BlockSpec((B, H), lambda i: (i, 0)),
        out_shape=jax.ShapeDtypeStruct((R, H), jnp.float32),
    )(a0, a1, hs, p0, p1, b)


def _tc_prep_dec(h2, W, q0, q1, B=640):
    Rc, H2 = h2.shape
    H = H2 // 2
    D = W.shape[1]

    def body(h_ref, w_ref, q0_ref, q1_ref, o_ref):
        degc = (q0_ref[:, 0:1] + q0_ref[:, 1:2]
                + q1_ref[:, 0:1] + q1_ref[:, 1:2] + 1.0)
        xc = 0.5 * (h_ref[:, :H] + h_ref[:, H:])
        o_ref[...] = jnp.dot(xc, w_ref[...],
                             preferred_element_type=jnp.float32) * lax.rsqrt(degc)

    return pl.pallas_call(
        body,
        grid=(Rc // B,),
        in_specs=[
            pl.BlockSpec((B, H2), lambda i: (i, 0)),
            pl.BlockSpec((H, D), lambda i: (0, 0)),
            pl.BlockSpec((B, 2), lambda i: (i, 0)),
            pl.BlockSpec((B, 2), lambda i: (i, 0)),
        ],
        out_specs=pl.BlockSpec((B, D), lambda i: (i, 0)),
        out_shape=jax.ShapeDtypeStruct((Rc, D), jnp.float32),
    )(h2, W, q0, q1)


def _tc_final(a0, a1, hds, q0, q1, b, B=640):
    Rc, D = hds.shape

    def body(a0_ref, a1_ref, hds_ref, q0_ref, q1_ref, b_ref, o_ref):
        degc = (q0_ref[:, 0:1] + q0_ref[:, 1:2]
                + q1_ref[:, 0:1] + q1_ref[:, 1:2] + 1.0)
        xd = ((a0_ref[...] + a1_ref[...] + hds_ref[...]) * lax.rsqrt(degc)
              + b_ref[...])
        o_ref[:, :D] = xd
        o_ref[:, D:] = xd

    return pl.pallas_call(
        body,
        grid=(Rc // B,),
        in_specs=[
            pl.BlockSpec((B, D), lambda i: (i, 0)),
            pl.BlockSpec((B, D), lambda i: (i, 0)),
            pl.BlockSpec((B, D), lambda i: (i, 0)),
            pl.BlockSpec((B, 2), lambda i: (i, 0)),
            pl.BlockSpec((B, 2), lambda i: (i, 0)),
            pl.BlockSpec((1, D), lambda i: (0, 0)),
        ],
        out_specs=pl.BlockSpec((B, 2 * D), lambda i: (i, 0)),
        out_shape=jax.ShapeDtypeStruct((Rc, 2 * D), jnp.float32),
    )(a0, a1, hds, q0, q1, b)


def kernel(x, edge_index, batch, W_enc, b_enc, W_dec, b_dec):
    N, D = x.shape
    H = W_enc.shape[1]
    E = edge_index.shape[1]
    Nc = N // 2

    Rc = ((Nc + 1 + 255) // 256) * 256
    R = 2 * Rc
    S = -(-(-(-E // CH)) // (NS * 8)) * 8

    def _splitn(frac, nb):
        C = -(-E // CH)
        c0 = -(-int(C * frac) // (NS * nb)) * nb
        c1 = max(nb, -(-(C - NS * c0) // (NS * nb)) * nb)
        return c0, c1

    c0f, c1f = _splitn(1.0, 4)
    c0c, c1c = _splitn(1.0, 3)
    iters = NS * S // NW

    C_pad = max(NS * S, NS * (c0f + c1f), NS * (c0c + c1c))
    pad_e = C_pad * CH - E
    src = jnp.concatenate(
        [edge_index[0], jnp.full((pad_e,), N, jnp.int32)]).reshape(-1, CH)
    dst = jnp.concatenate(
        [edge_index[1], jnp.full((pad_e,), N, jnp.int32)]).reshape(-1, CH)

    zeros8 = jnp.zeros((R, 8), jnp.float32)
    ones8 = jnp.zeros((CH, 8), jnp.float32).at[:, 0].set(1.0)
    zf = jnp.zeros((CH, H), jnp.float32)
    zc = jnp.zeros((CH, D), jnp.float32)
    x_pad = jnp.concatenate([x, jnp.zeros((R - N, D), x.dtype)])

    degp = _sc_degree(dst, zeros8, ones8, R, iters)
    p0 = degp[0, :, :1]
    p1 = degp[1, :, :1]

    hs = _tc_prep_enc(x_pad, W_enc, p0, p1)
    accf = _sc_edge_pass(src, dst, hs, zf, R, H, False, c0f, c1f, 4)
    h_enc = _tc_post_enc(accf[0], accf[1], hs, p0, p1, b_enc.reshape(1, H))

    q0 = p0.reshape(Rc, 2)
    q1 = p1.reshape(Rc, 2)
    hds = _tc_prep_dec(h_enc.reshape(Rc, 2 * H), W_dec, q0, q1)
    accc = _sc_edge_pass(src, dst, hds, zc, Rc, D, True, c0c, c1c, 3)
    outd = _tc_final(accc[0], accc[1], hds, q0, q1, b_dec.reshape(1, D))

    return outd[:Nc].reshape(N, D)


# speedup vs baseline: 2.0244x; 2.0244x over previous
import functools

import jax
import jax.numpy as jnp
from jax import lax
from jax.experimental import pallas as pl
from jax.experimental.pallas import tpu as pltpu
from jax.experimental.pallas import tpu_sc as plsc

NC = 2
NS = 16
NW = NC * NS
CH = 128
NB = 4

_SC_PARAMS = pltpu.CompilerParams(use_tc_tiling_on_sc=False)


def _sc_degree(dst2, zeros8, ones8, R, iters):
    rpt = R // NS
    mesh = plsc.VectorSubcoreMesh(core_axis_name="c", subcore_axis_name="s")
    K = 8
    rounds = iters // K

    @functools.partial(
        pl.kernel,
        out_type=jax.ShapeDtypeStruct((NC, R, 8), jnp.float32),
        mesh=mesh,
        scratch_types=[
            pltpu.VMEM((iters, CH), jnp.int32),
            pltpu.VMEM((CH, 8), jnp.float32),
            pltpu.VMEM((rpt, 8), jnp.float32),
            pltpu.VMEM_SHARED((R, 8), jnp.float32),
            pltpu.SemaphoreType.DMA,
        ],
        compiler_params=_SC_PARAMS,
    )
    def k(dst_hbm, zeros_hbm, ones_hbm, out_hbm, idx_v, ones_v, chunk_v,
          hist, sem):
        cid = lax.axis_index("c")
        sid = lax.axis_index("s")
        wid = sid * NC + cid
        row = pl.ds(sid * rpt, rpt)
        pltpu.sync_copy(zeros_hbm.at[row], chunk_v)
        pltpu.sync_copy(chunk_v, hist.at[row])
        pltpu.sync_copy(dst_hbm.at[pl.ds(wid * iters, iters)], idx_v)
        pltpu.sync_copy(ones_hbm, ones_v)
        plsc.subcore_barrier()

        def body(g, carry):
            for b in range(K):
                pltpu.async_copy(
                    ones_v, hist.at[idx_v.at[g * K + b]], sem, add=True)
            for b in range(K):
                pltpu.make_async_copy(
                    ones_v, hist.at[idx_v.at[0]], sem).wait()
            return carry

        lax.fori_loop(0, rounds, body, 0)
        plsc.subcore_barrier()
        pltpu.sync_copy(hist.at[row], chunk_v)
        pltpu.sync_copy(chunk_v, out_hbm.at[cid, row])

    return k(dst2, zeros8, ones8)


def _sc_edge_pass(src2, dst2, table, zeros, R, W, shift, c0, c1, nb):
    rpt = R // NS
    mesh = plsc.VectorSubcoreMesh(core_axis_name="c", subcore_axis_name="s")
    cmax = max(c0, c1)
    NB = nb

    @functools.partial(
        pl.kernel,
        out_type=jax.ShapeDtypeStruct((NC, R, W), jnp.float32),
        mesh=mesh,
        scratch_types=(
            [pltpu.VMEM((cmax, CH), jnp.int32),
             pltpu.VMEM((cmax, CH), jnp.int32)]
            + [pltpu.VMEM((CH, W), jnp.float32) for _ in range(NB)]
            + [pltpu.VMEM_SHARED((R, W), jnp.float32)]
            + [pltpu.SemaphoreType.DMA for _ in range(2 * NB)]
        ),
        compiler_params=_SC_PARAMS,
    )
    def k(src_hbm, dst_hbm, table_hbm, zeros_hbm, out_hbm,
          idxs_v, idxd_v, *bufs_and_sems):
        rows = bufs_and_sems[:NB]
        acc = bufs_and_sems[NB]
        semg = bufs_and_sems[NB + 1:NB + 1 + NB]
        sems = bufs_and_sems[NB + 1 + NB:]
        cid = lax.axis_index("c")
        sid = lax.axis_index("s")
        row = pl.ds(sid * rpt, rpt)

        scope = jax.named_scope
        chunks = []
        o = 0
        while o < rpt:
            c = min(CH, rpt - o)
            chunks.append((o, c))
            o += c
        with scope("ph_init"):
            pltpu.sync_copy(zeros_hbm, rows[0])
            for (o, c) in chunks:
                pltpu.sync_copy(rows[0].at[pl.ds(0, c)],
                                acc.at[pl.ds(sid * rpt + o, c)])

        @pl.when(cid == 0)
        def _load0():
            pltpu.sync_copy(src_hbm.at[pl.ds(sid * c0, c0)],
                            idxs_v.at[pl.ds(0, c0)])
            pltpu.sync_copy(dst_hbm.at[pl.ds(sid * c0, c0)],
                            idxd_v.at[pl.ds(0, c0)])

        @pl.when(cid == 1)
        def _load1():
            pltpu.sync_copy(src_hbm.at[pl.ds(NS * c0 + sid * c1, c1)],
                            idxs_v.at[pl.ds(0, c1)])
            pltpu.sync_copy(dst_hbm.at[pl.ds(NS * c0 + sid * c1, c1)],
                            idxd_v.at[pl.ds(0, c1)])

        rounds = jnp.where(cid == 0, c0 // NB, c1 // NB)
        if shift:
            def sh(i, carry):
                for j in range(CH // 16):
                    sl = pl.ds(j * 16, 16)
                    idxs_v[i, sl] = idxs_v[i, sl] >> 1
                    idxd_v[i, sl] = idxd_v[i, sl] >> 1
                return carry
            lax.fori_loop(0, rounds * NB, sh, 0)
        plsc.subcore_barrier()

        def body(g, carry):
            for b in range(NB):
                @pl.when(g > 0)
                def _drain():
                    pltpu.make_async_copy(
                        rows[b], acc.at[idxd_v.at[0]], sems[b]).wait()
                pltpu.async_copy(
                    table_hbm.at[idxs_v.at[g * NB + b]], rows[b], semg[b])
            for b in range(NB):
                pltpu.make_async_copy(
                    table_hbm.at[idxs_v.at[0]], rows[b], semg[b]).wait()
                pltpu.async_copy(
                    rows[b], acc.at[idxd_v.at[g * NB + b]], sems[b],
                    add=True)
            return carry

        with scope("ph_loop"):
            lax.fori_loop(0, rounds, body, 0)
            for b in range(NB):
                pltpu.make_async_copy(
                    rows[b], acc.at[idxd_v.at[0]], sems[b]).wait()
            plsc.subcore_barrier()
        scope2 = jax.named_scope("ph_out")
        scope2.__enter__()
        live = {}
        for z, (o, c) in enumerate(chunks):
            sl = pl.ds(sid * rpt + o, c)
            b = z % NB
            if b in live:
                pltpu.make_async_copy(
                    rows[b].at[pl.ds(0, live[b])],
                    out_hbm.at[cid, pl.ds(0, live[b])], semg[b]).wait()
            pltpu.sync_copy(acc.at[sl], rows[b].at[pl.ds(0, c)])
            pltpu.async_copy(rows[b].at[pl.ds(0, c)],
                             out_hbm.at[cid, sl], semg[b])
            live[b] = c
        for b, c in live.items():
            pltpu.make_async_copy(
                rows[b].at[pl.ds(0, c)],
                out_hbm.at[cid, pl.ds(0, c)], semg[b]).wait()
        scope2.__exit__(None, None, None)

    return k(src2, dst2, table, zeros)


def _sc_edge_pass2(srcp, dstp, table, zeros, R, W, shift, cpt, nb):
    rpt = R // NS
    mesh = plsc.VectorSubcoreMesh(core_axis_name="c", subcore_axis_name="s")
    NB = nb
    CH2 = CH // 2

    @functools.partial(
        pl.kernel,
        out_type=jax.ShapeDtypeStruct((NC, R, W), jnp.float32),
        mesh=mesh,
        scratch_types=(
            [pltpu.VMEM((cpt, CH2), jnp.int32),
             pltpu.VMEM((cpt, CH2), jnp.int32),
             pltpu.VMEM((NB, CH), jnp.int32),
             pltpu.VMEM((NB, CH), jnp.int32)]
            + [pltpu.VMEM((CH, W), jnp.float32) for _ in range(NB)]
            + [pltpu.VMEM_SHARED((R, W), jnp.float32),
               pltpu.VMEM_SHARED((R, W), jnp.float32)]
            + [pltpu.SemaphoreType.DMA for _ in range(2 * NB)]
        ),
        compiler_params=_SC_PARAMS,
    )
    def k(src_hbm, dst_hbm, table_hbm, zeros_hbm, out_hbm,
          idxsp, idxdp, idxs32, idxd32, *bufs_and_sems):
        rows = bufs_and_sems[:NB]
        acc = bufs_and_sems[NB]
        table_sh = bufs_and_sems[NB + 1]
        semg = bufs_and_sems[NB + 2:NB + 2 + NB]
        sems = bufs_and_sems[NB + 2 + NB:]
        cid = lax.axis_index("c")
        sid = lax.axis_index("s")
        wid = sid * NC + cid
        row = pl.ds(sid * rpt, rpt)

        chunks = []
        o = 0
        while o < rpt:
            c = min(CH, rpt - o)
            chunks.append((o, c))
            o += c
        pltpu.sync_copy(table_hbm.at[row], table_sh.at[row])
        pltpu.sync_copy(zeros_hbm, rows[0])
        for (o, c) in chunks:
            pltpu.sync_copy(rows[0].at[pl.ds(0, c)],
                            acc.at[pl.ds(sid * rpt + o, c)])
        pltpu.sync_copy(src_hbm.at[pl.ds(wid * cpt, cpt)], idxsp)
        pltpu.sync_copy(dst_hbm.at[pl.ds(wid * cpt, cpt)], idxdp)
        plsc.subcore_barrier()

        def widen(packed, out_ref, b, i):
            for j in range(CH2 // 16):
                v = packed[i, pl.ds(j * 16, 16)]
                lo = v & 0xFFFF
                hi = v >> 16
                if shift:
                    lo = lo >> 1
                    hi = hi >> 1
                out_ref[b, pl.ds(j * 32, 16)] = lo
                out_ref[b, pl.ds(j * 32 + 16, 16)] = hi

        def body(g, carry):
            for b in range(NB):
                @pl.when(g > 0)
                def _drain():
                    pltpu.make_async_copy(
                        rows[b], acc.at[idxd32.at[0]], sems[b]).wait()
                widen(idxsp, idxs32, b, g * NB + b)
                pltpu.async_copy(
                    table_sh.at[idxs32.at[b]], rows[b], semg[b])
            for b in range(NB):
                pltpu.make_async_copy(
                    table_sh.at[idxs32.at[0]], rows[b], semg[b]).wait()
                widen(idxdp, idxd32, b, g * NB + b)
                pltpu.async_copy(
                    rows[b], acc.at[idxd32.at[b]], sems[b], add=True)
            return carry

        lax.fori_loop(0, cpt // NB, body, 0)
        for b in range(NB):
            pltpu.make_async_copy(
                rows[b], acc.at[idxd32.at[0]], sems[b]).wait()
        plsc.subcore_barrier()
        live = {}
        for z, (o, c) in enumerate(chunks):
            sl = pl.ds(sid * rpt + o, c)
            b = z % NB
            if b in live:
                pltpu.make_async_copy(
                    rows[b].at[pl.ds(0, live[b])],
                    out_hbm.at[cid, pl.ds(0, live[b])], semg[b]).wait()
            pltpu.sync_copy(acc.at[sl], rows[b].at[pl.ds(0, c)])
            pltpu.async_copy(rows[b].at[pl.ds(0, c)],
                             out_hbm.at[cid, sl], semg[b])
            live[b] = c
        for b, c in live.items():
            pltpu.make_async_copy(
                rows[b].at[pl.ds(0, c)],
                out_hbm.at[cid, pl.ds(0, c)], semg[b]).wait()

    return k(srcp, dstp, table, zeros)


def _tc_prep_enc(x_pad, W, p0, p1, B=640):
    R, D = x_pad.shape
    H = W.shape[1]

    def body(x_ref, w_ref, p0_ref, p1_ref, o_ref):
        dinv = lax.rsqrt(p0_ref[...] + p1_ref[...] + 1.0)
        o_ref[...] = jnp.dot(x_ref[...], w_ref[...],
                             preferred_element_type=jnp.float32) * dinv

    return pl.pallas_call(
        body,
        grid=(R // B,),
        in_specs=[
            pl.BlockSpec((B, D), lambda i: (i, 0)),
            pl.BlockSpec((D, H), lambda i: (0, 0)),
            pl.BlockSpec((B, 1), lambda i: (i, 0)),
            pl.BlockSpec((B, 1), lambda i: (i, 0)),
        ],
        out_specs=pl.BlockSpec((B, H), lambda i: (i, 0)),
        out_shape=jax.ShapeDtypeStruct((R, H), jnp.float32),
    )(x_pad, W, p0, p1)


def _tc_post_enc(a0, a1, hs, p0, p1, b, B=640):
    R, H = hs.shape

    def body(a0_ref, a1_ref, hs_ref, p0_ref, p1_ref, b_ref, o_ref):
        dinv = lax.rsqrt(p0_ref[...] + p1_ref[...] + 1.0)
        s = (a0_ref[...] + a1_ref[...] + hs_ref[...]) * dinv + b_ref[...]
        o_ref[...] = jnp.maximum(s, 0.0)

    return pl.pallas_call(
        body,
        grid=(R // B,),
        in_specs=[
            pl.BlockSpec((B, H), lambda i: (i, 0)),
            pl.BlockSpec((B, H), lambda i: (i, 0)),
            pl.BlockSpec((B, H), lambda i: (i, 0)),
            pl.BlockSpec((B, 1), lambda i: (i, 0)),
            pl.BlockSpec((B, 1), lambda i: (i, 0)),
            pl.BlockSpec((1, H), lambda i: (0, 0)),
        ],
        out_specs=pl.BlockSpec((B, H), lambda i: (i, 0)),
        out_shape=jax.ShapeDtypeStruct((R, H), jnp.float32),
    )(a0, a1, hs, p0, p1, b)


def _tc_prep_dec(h2, W, q0, q1, B=640):
    Rc, H2 = h2.shape
    H = H2 // 2
    D = W.shape[1]

    def body(h_ref, w_ref, q0_ref, q1_ref, o_ref):
        degc = (q0_ref[:, 0:1] + q0_ref[:, 1:2]
                + q1_ref[:, 0:1] + q1_ref[:, 1:2] + 1.0)
        xc = 0.5 * (h_ref[:, :H] + h_ref[:, H:])
        o_ref[...] = jnp.dot(xc, w_ref[...],
                             preferred_element_type=jnp.float32) * lax.rsqrt(degc)

    return pl.pallas_call(
        body,
        grid=(Rc // B,),
        in_specs=[
            pl.BlockSpec((B, H2), lambda i: (i, 0)),
            pl.BlockSpec((H, D), lambda i: (0, 0)),
            pl.BlockSpec((B, 2), lambda i: (i, 0)),
            pl.BlockSpec((B, 2), lambda i: (i, 0)),
        ],
        out_specs=pl.BlockSpec((B, D), lambda i: (i, 0)),
        out_shape=jax.ShapeDtypeStruct((Rc, D), jnp.float32),
    )(h2, W, q0, q1)


def _tc_final(a0, a1, hds, q0, q1, b, B=640):
    Rc, D = hds.shape

    def body(a0_ref, a1_ref, hds_ref, q0_ref, q1_ref, b_ref, o_ref):
        degc = (q0_ref[:, 0:1] + q0_ref[:, 1:2]
                + q1_ref[:, 0:1] + q1_ref[:, 1:2] + 1.0)
        xd = ((a0_ref[...] + a1_ref[...] + hds_ref[...]) * lax.rsqrt(degc)
              + b_ref[...])
        o_ref[:, :D] = xd
        o_ref[:, D:] = xd

    return pl.pallas_call(
        body,
        grid=(Rc // B,),
        in_specs=[
            pl.BlockSpec((B, D), lambda i: (i, 0)),
            pl.BlockSpec((B, D), lambda i: (i, 0)),
            pl.BlockSpec((B, D), lambda i: (i, 0)),
            pl.BlockSpec((B, 2), lambda i: (i, 0)),
            pl.BlockSpec((B, 2), lambda i: (i, 0)),
            pl.BlockSpec((1, D), lambda i: (0, 0)),
        ],
        out_specs=pl.BlockSpec((B, 2 * D), lambda i: (i, 0)),
        out_shape=jax.ShapeDtypeStruct((Rc, 2 * D), jnp.float32),
    )(a0, a1, hds, q0, q1, b)


def kernel(x, edge_index, batch, W_enc, b_enc, W_dec, b_dec):
    N, D = x.shape
    H = W_enc.shape[1]
    E = edge_index.shape[1]
    Nc = N // 2

    Rc = ((Nc + 1 + 255) // 256) * 256
    R = 2 * Rc
    S = -(-(-(-E // CH)) // (NS * 8)) * 8

    def _splitn(frac, nb):
        C = -(-E // CH)
        c0 = -(-int(C * frac) // (NS * nb)) * nb
        c1 = max(nb, -(-(C - NS * c0) // (NS * nb)) * nb)
        return c0, c1

    cpt = -(-S // NC)
    cpt = -(-cpt // 4) * 4
    iters = NS * S // NW

    C_pad = max(NS * S, NW * cpt)
    pad_e = C_pad * CH - E
    src = jnp.concatenate(
        [edge_index[0], jnp.full((pad_e,), N, jnp.int32)]).reshape(-1, CH)
    dst = jnp.concatenate(
        [edge_index[1], jnp.full((pad_e,), N, jnp.int32)]).reshape(-1, CH)
    srcp = lax.bitcast_convert_type(
        src.astype(jnp.int16).reshape(-1, 2), jnp.int32).reshape(C_pad, CH // 2)
    dstp = lax.bitcast_convert_type(
        dst.astype(jnp.int16).reshape(-1, 2), jnp.int32).reshape(C_pad, CH // 2)

    zeros8 = jnp.zeros((R, 8), jnp.float32)
    ones8 = jnp.zeros((CH, 8), jnp.float32).at[:, 0].set(1.0)
    zf = jnp.zeros((CH, H), jnp.float32)
    zc = jnp.zeros((CH, D), jnp.float32)
    x_pad = jnp.concatenate([x, jnp.zeros((R - N, D), x.dtype)])

    degp = _sc_degree(dst, zeros8, ones8, R, iters)
    p0 = degp[0, :, :1]
    p1 = degp[1, :, :1]

    hs = _tc_prep_enc(x_pad, W_enc, p0, p1)
    accf = _sc_edge_pass2(srcp, dstp, hs, zf, R, H, False, cpt, 4)
    h_enc = _tc_post_enc(accf[0], accf[1], hs, p0, p1, b_enc.reshape(1, H))

    q0 = p0.reshape(Rc, 2)
    q1 = p1.reshape(Rc, 2)
    hds = _tc_prep_dec(h_enc.reshape(Rc, 2 * H), W_dec, q0, q1)
    accc = _sc_edge_pass2(srcp, dstp, hds, zc, Rc, D, True, cpt, 2)
    outd = _tc_final(accc[0], accc[1], hds, q0, q1, b_dec.reshape(1, D))

    return outd[:Nc].reshape(N, D)


# speedup vs baseline: 3.1853x; 1.5735x over previous
import functools

import jax
import jax.numpy as jnp
from jax import lax
from jax.experimental import pallas as pl
from jax.experimental.pallas import tpu as pltpu
from jax.experimental.pallas import tpu_sc as plsc

NC = 2
NS = 16
NW = NC * NS
CH = 128
NB = 4

_SC_PARAMS = pltpu.CompilerParams(use_tc_tiling_on_sc=False)


def _sc_degree(dst2, zeros8, ones8, R, iters):
    rpt = R // NS
    mesh = plsc.VectorSubcoreMesh(core_axis_name="c", subcore_axis_name="s")
    K = 8
    rounds = iters // K

    @functools.partial(
        pl.kernel,
        out_type=jax.ShapeDtypeStruct((NC, R, 8), jnp.float32),
        mesh=mesh,
        scratch_types=[
            pltpu.VMEM((iters, CH), jnp.int32),
            pltpu.VMEM((CH, 8), jnp.float32),
            pltpu.VMEM((rpt, 8), jnp.float32),
            pltpu.VMEM_SHARED((R, 8), jnp.float32),
            pltpu.SemaphoreType.DMA,
        ],
        compiler_params=_SC_PARAMS,
    )
    def k(dst_hbm, zeros_hbm, ones_hbm, out_hbm, idx_v, ones_v, chunk_v,
          hist, sem):
        cid = lax.axis_index("c")
        sid = lax.axis_index("s")
        wid = sid * NC + cid
        row = pl.ds(sid * rpt, rpt)
        pltpu.sync_copy(zeros_hbm.at[row], chunk_v)
        pltpu.sync_copy(chunk_v, hist.at[row])
        pltpu.sync_copy(dst_hbm.at[pl.ds(wid * iters, iters)], idx_v)
        pltpu.sync_copy(ones_hbm, ones_v)
        plsc.subcore_barrier()

        def body(g, carry):
            for b in range(K):
                pltpu.async_copy(
                    ones_v, hist.at[idx_v.at[g * K + b]], sem, add=True)
            for b in range(K):
                pltpu.make_async_copy(
                    ones_v, hist.at[idx_v.at[0]], sem).wait()
            return carry

        lax.fori_loop(0, rounds, body, 0)
        plsc.subcore_barrier()
        pltpu.sync_copy(hist.at[row], chunk_v)
        pltpu.sync_copy(chunk_v, out_hbm.at[cid, row])

    return k(dst2, zeros8, ones8)


def _sc_edge_pass(src2, dst2, table, zeros, R, W, shift, c0, c1, nb):
    rpt = R // NS
    mesh = plsc.VectorSubcoreMesh(core_axis_name="c", subcore_axis_name="s")
    cmax = max(c0, c1)
    NB = nb

    @functools.partial(
        pl.kernel,
        out_type=jax.ShapeDtypeStruct((NC, R, W), jnp.float32),
        mesh=mesh,
        scratch_types=(
            [pltpu.VMEM((cmax, CH), jnp.int32),
             pltpu.VMEM((cmax, CH), jnp.int32)]
            + [pltpu.VMEM((CH, W), jnp.float32) for _ in range(NB)]
            + [pltpu.VMEM_SHARED((R, W), jnp.float32)]
            + [pltpu.SemaphoreType.DMA for _ in range(2 * NB)]
        ),
        compiler_params=_SC_PARAMS,
    )
    def k(src_hbm, dst_hbm, table_hbm, zeros_hbm, out_hbm,
          idxs_v, idxd_v, *bufs_and_sems):
        rows = bufs_and_sems[:NB]
        acc = bufs_and_sems[NB]
        semg = bufs_and_sems[NB + 1:NB + 1 + NB]
        sems = bufs_and_sems[NB + 1 + NB:]
        cid = lax.axis_index("c")
        sid = lax.axis_index("s")
        row = pl.ds(sid * rpt, rpt)

        scope = jax.named_scope
        chunks = []
        o = 0
        while o < rpt:
            c = min(CH, rpt - o)
            chunks.append((o, c))
            o += c
        with scope("ph_init"):
            pltpu.sync_copy(zeros_hbm, rows[0])
            for (o, c) in chunks:
                pltpu.sync_copy(rows[0].at[pl.ds(0, c)],
                                acc.at[pl.ds(sid * rpt + o, c)])

        @pl.when(cid == 0)
        def _load0():
            pltpu.sync_copy(src_hbm.at[pl.ds(sid * c0, c0)],
                            idxs_v.at[pl.ds(0, c0)])
            pltpu.sync_copy(dst_hbm.at[pl.ds(sid * c0, c0)],
                            idxd_v.at[pl.ds(0, c0)])

        @pl.when(cid == 1)
        def _load1():
            pltpu.sync_copy(src_hbm.at[pl.ds(NS * c0 + sid * c1, c1)],
                            idxs_v.at[pl.ds(0, c1)])
            pltpu.sync_copy(dst_hbm.at[pl.ds(NS * c0 + sid * c1, c1)],
                            idxd_v.at[pl.ds(0, c1)])

        rounds = jnp.where(cid == 0, c0 // NB, c1 // NB)
        if shift:
            def sh(i, carry):
                for j in range(CH // 16):
                    sl = pl.ds(j * 16, 16)
                    idxs_v[i, sl] = idxs_v[i, sl] >> 1
                    idxd_v[i, sl] = idxd_v[i, sl] >> 1
                return carry
            lax.fori_loop(0, rounds * NB, sh, 0)
        plsc.subcore_barrier()

        def body(g, carry):
            for b in range(NB):
                @pl.when(g > 0)
                def _drain():
                    pltpu.make_async_copy(
                        rows[b], acc.at[idxd_v.at[0]], sems[b]).wait()
                pltpu.async_copy(
                    table_hbm.at[idxs_v.at[g * NB + b]], rows[b], semg[b])
            for b in range(NB):
                pltpu.make_async_copy(
                    table_hbm.at[idxs_v.at[0]], rows[b], semg[b]).wait()
                pltpu.async_copy(
                    rows[b], acc.at[idxd_v.at[g * NB + b]], sems[b],
                    add=True)
            return carry

        with scope("ph_loop"):
            lax.fori_loop(0, rounds, body, 0)
            for b in range(NB):
                pltpu.make_async_copy(
                    rows[b], acc.at[idxd_v.at[0]], sems[b]).wait()
            plsc.subcore_barrier()
        scope2 = jax.named_scope("ph_out")
        scope2.__enter__()
        live = {}
        for z, (o, c) in enumerate(chunks):
            sl = pl.ds(sid * rpt + o, c)
            b = z % NB
            if b in live:
                pltpu.make_async_copy(
                    rows[b].at[pl.ds(0, live[b])],
                    out_hbm.at[cid, pl.ds(0, live[b])], semg[b]).wait()
            pltpu.sync_copy(acc.at[sl], rows[b].at[pl.ds(0, c)])
            pltpu.async_copy(rows[b].at[pl.ds(0, c)],
                             out_hbm.at[cid, sl], semg[b])
            live[b] = c
        for b, c in live.items():
            pltpu.make_async_copy(
                rows[b].at[pl.ds(0, c)],
                out_hbm.at[cid, pl.ds(0, c)], semg[b]).wait()
        scope2.__exit__(None, None, None)

    return k(src2, dst2, table, zeros)


def _sc_edge_pass2(srcp, dstp, table, zeros, R, W, shift, cpt, nb):
    rpt = R // NS
    mesh = plsc.VectorSubcoreMesh(core_axis_name="c", subcore_axis_name="s")
    NB = nb
    CH2 = CH // 2

    @functools.partial(
        pl.kernel,
        out_type=jax.ShapeDtypeStruct((NC, R, W), jnp.float32),
        mesh=mesh,
        scratch_types=(
            [pltpu.VMEM((cpt, CH2), jnp.int32),
             pltpu.VMEM((cpt, CH2), jnp.int32),
             pltpu.VMEM((NB, CH), jnp.int32),
             pltpu.VMEM((NB, CH), jnp.int32)]
            + [pltpu.VMEM((CH, W), jnp.float32) for _ in range(NB)]
            + [pltpu.VMEM_SHARED((R, W), jnp.float32),
               pltpu.VMEM_SHARED((R, W), jnp.float32)]
            + [pltpu.SemaphoreType.DMA for _ in range(2 * NB)]
        ),
        compiler_params=_SC_PARAMS,
    )
    def k(src_hbm, dst_hbm, table_hbm, zeros_hbm, out_hbm,
          idxsp, idxdp, idxs32, idxd32, *bufs_and_sems):
        rows = bufs_and_sems[:NB]
        acc = bufs_and_sems[NB]
        table_sh = bufs_and_sems[NB + 1]
        semg = bufs_and_sems[NB + 2:NB + 2 + NB]
        sems = bufs_and_sems[NB + 2 + NB:]
        cid = lax.axis_index("c")
        sid = lax.axis_index("s")
        wid = sid * NC + cid
        row = pl.ds(sid * rpt, rpt)

        chunks = []
        o = 0
        while o < rpt:
            c = min(CH, rpt - o)
            chunks.append((o, c))
            o += c
        pltpu.sync_copy(table_hbm.at[row], table_sh.at[row])
        pltpu.sync_copy(zeros_hbm, rows[0])
        for (o, c) in chunks:
            pltpu.sync_copy(rows[0].at[pl.ds(0, c)],
                            acc.at[pl.ds(sid * rpt + o, c)])
        pltpu.sync_copy(src_hbm.at[pl.ds(wid * cpt, cpt)], idxsp)
        pltpu.sync_copy(dst_hbm.at[pl.ds(wid * cpt, cpt)], idxdp)
        plsc.subcore_barrier()

        def widen(packed, out_ref, b, i):
            for j in range(CH2 // 16):
                v = packed[i, pl.ds(j * 16, 16)]
                lo = v & 0xFFFF
                hi = v >> 16
                if shift:
                    lo = lo >> 1
                    hi = hi >> 1
                out_ref[b, pl.ds(j * 16, 16)] = lo
                out_ref[b, pl.ds(CH2 + j * 16, 16)] = hi

        def body(g, carry):
            for b in range(NB):
                @pl.when(g > 0)
                def _drain():
                    pltpu.make_async_copy(
                        rows[b], acc.at[idxd32.at[0]], sems[b]).wait()
                widen(idxsp, idxs32, b, g * NB + b)
                pltpu.async_copy(
                    table_sh.at[idxs32.at[b]], rows[b], semg[b])
            for b in range(NB):
                pltpu.make_async_copy(
                    table_sh.at[idxs32.at[0]], rows[b], semg[b]).wait()
                widen(idxdp, idxd32, b, g * NB + b)
                pltpu.async_copy(
                    rows[b], acc.at[idxd32.at[b]], sems[b], add=True)
            return carry

        lax.fori_loop(0, cpt // NB, body, 0)
        for b in range(NB):
            pltpu.make_async_copy(
                rows[b], acc.at[idxd32.at[0]], sems[b]).wait()
        plsc.subcore_barrier()
        live = {}
        for z, (o, c) in enumerate(chunks):
            sl = pl.ds(sid * rpt + o, c)
            b = z % NB
            if b in live:
                pltpu.make_async_copy(
                    rows[b].at[pl.ds(0, live[b])],
                    out_hbm.at[cid, pl.ds(0, live[b])], semg[b]).wait()
            pltpu.sync_copy(acc.at[sl], rows[b].at[pl.ds(0, c)])
            pltpu.async_copy(rows[b].at[pl.ds(0, c)],
                             out_hbm.at[cid, sl], semg[b])
            live[b] = c
        for b, c in live.items():
            pltpu.make_async_copy(
                rows[b].at[pl.ds(0, c)],
                out_hbm.at[cid, pl.ds(0, c)], semg[b]).wait()

    return k(srcp, dstp, table, zeros)


def _tc_prep_enc(x_pad, W, p0, p1, B=640):
    R, D = x_pad.shape
    H = W.shape[1]

    def body(x_ref, w_ref, p0_ref, p1_ref, o_ref):
        dinv = lax.rsqrt(p0_ref[...] + p1_ref[...] + 1.0)
        o_ref[...] = jnp.dot(x_ref[...], w_ref[...],
                             preferred_element_type=jnp.float32) * dinv

    return pl.pallas_call(
        body,
        grid=(R // B,),
        in_specs=[
            pl.BlockSpec((B, D), lambda i: (i, 0)),
            pl.BlockSpec((D, H), lambda i: (0, 0)),
            pl.BlockSpec((B, 1), lambda i: (i, 0)),
            pl.BlockSpec((B, 1), lambda i: (i, 0)),
        ],
        out_specs=pl.BlockSpec((B, H), lambda i: (i, 0)),
        out_shape=jax.ShapeDtypeStruct((R, H), jnp.float32),
    )(x_pad, W, p0, p1)


def _tc_post_enc(a0, a1, hs, p0, p1, b, B=640):
    R, H = hs.shape

    def body(a0_ref, a1_ref, hs_ref, p0_ref, p1_ref, b_ref, o_ref):
        dinv = lax.rsqrt(p0_ref[...] + p1_ref[...] + 1.0)
        s = (a0_ref[...] + a1_ref[...] + hs_ref[...]) * dinv + b_ref[...]
        o_ref[...] = jnp.maximum(s, 0.0)

    return pl.pallas_call(
        body,
        grid=(R // B,),
        in_specs=[
            pl.BlockSpec((B, H), lambda i: (i, 0)),
            pl.BlockSpec((B, H), lambda i: (i, 0)),
            pl.BlockSpec((B, H), lambda i: (i, 0)),
            pl.BlockSpec((B, 1), lambda i: (i, 0)),
            pl.BlockSpec((B, 1), lambda i: (i, 0)),
            pl.BlockSpec((1, H), lambda i: (0, 0)),
        ],
        out_specs=pl.BlockSpec((B, H), lambda i: (i, 0)),
        out_shape=jax.ShapeDtypeStruct((R, H), jnp.float32),
    )(a0, a1, hs, p0, p1, b)


def _tc_prep_dec(h2, W, q0, q1, B=640):
    Rc, H2 = h2.shape
    H = H2 // 2
    D = W.shape[1]

    def body(h_ref, w_ref, q0_ref, q1_ref, o_ref):
        degc = (q0_ref[:, 0:1] + q0_ref[:, 1:2]
                + q1_ref[:, 0:1] + q1_ref[:, 1:2] + 1.0)
        xc = 0.5 * (h_ref[:, :H] + h_ref[:, H:])
        o_ref[...] = jnp.dot(xc, w_ref[...],
                             preferred_element_type=jnp.float32) * lax.rsqrt(degc)

    return pl.pallas_call(
        body,
        grid=(Rc // B,),
        in_specs=[
            pl.BlockSpec((B, H2), lambda i: (i, 0)),
            pl.BlockSpec((H, D), lambda i: (0, 0)),
            pl.BlockSpec((B, 2), lambda i: (i, 0)),
            pl.BlockSpec((B, 2), lambda i: (i, 0)),
        ],
        out_specs=pl.BlockSpec((B, D), lambda i: (i, 0)),
        out_shape=jax.ShapeDtypeStruct((Rc, D), jnp.float32),
    )(h2, W, q0, q1)


def _tc_final(a0, a1, hds, q0, q1, b, B=640):
    Rc, D = hds.shape

    def body(a0_ref, a1_ref, hds_ref, q0_ref, q1_ref, b_ref, o_ref):
        degc = (q0_ref[:, 0:1] + q0_ref[:, 1:2]
                + q1_ref[:, 0:1] + q1_ref[:, 1:2] + 1.0)
        xd = ((a0_ref[...] + a1_ref[...] + hds_ref[...]) * lax.rsqrt(degc)
              + b_ref[...])
        o_ref[:, :D] = xd
        o_ref[:, D:] = xd

    return pl.pallas_call(
        body,
        grid=(Rc // B,),
        in_specs=[
            pl.BlockSpec((B, D), lambda i: (i, 0)),
            pl.BlockSpec((B, D), lambda i: (i, 0)),
            pl.BlockSpec((B, D), lambda i: (i, 0)),
            pl.BlockSpec((B, 2), lambda i: (i, 0)),
            pl.BlockSpec((B, 2), lambda i: (i, 0)),
            pl.BlockSpec((1, D), lambda i: (0, 0)),
        ],
        out_specs=pl.BlockSpec((B, 2 * D), lambda i: (i, 0)),
        out_shape=jax.ShapeDtypeStruct((Rc, 2 * D), jnp.float32),
    )(a0, a1, hds, q0, q1, b)


def kernel(x, edge_index, batch, W_enc, b_enc, W_dec, b_dec):
    N, D = x.shape
    H = W_enc.shape[1]
    E = edge_index.shape[1]
    Nc = N // 2

    Rc = ((Nc + 1 + 255) // 256) * 256
    R = 2 * Rc
    S = -(-(-(-E // CH)) // (NS * 8)) * 8

    def _splitn(frac, nb):
        C = -(-E // CH)
        c0 = -(-int(C * frac) // (NS * nb)) * nb
        c1 = max(nb, -(-(C - NS * c0) // (NS * nb)) * nb)
        return c0, c1

    cpt = -(-S // NC)
    cpt = -(-cpt // 4) * 4
    iters = NS * S // NW

    C_pad = max(NS * S, NW * cpt)
    pad_e = C_pad * CH - E
    src = jnp.concatenate(
        [edge_index[0], jnp.full((pad_e,), N, jnp.int32)]).reshape(-1, CH)
    dst = jnp.concatenate(
        [edge_index[1], jnp.full((pad_e,), N, jnp.int32)]).reshape(-1, CH)
    srcp = src[:, :CH // 2] | (src[:, CH // 2:] << 16)
    dstp = dst[:, :CH // 2] | (dst[:, CH // 2:] << 16)

    zeros8 = jnp.zeros((R, 8), jnp.float32)
    ones8 = jnp.zeros((CH, 8), jnp.float32).at[:, 0].set(1.0)
    zf = jnp.zeros((CH, H), jnp.float32)
    zc = jnp.zeros((CH, D), jnp.float32)
    x_pad = jnp.concatenate([x, jnp.zeros((R - N, D), x.dtype)])

    degp = _sc_degree(dst, zeros8, ones8, R, iters)
    p0 = degp[0, :, :1]
    p1 = degp[1, :, :1]

    hs = _tc_prep_enc(x_pad, W_enc, p0, p1)
    accf = _sc_edge_pass2(srcp, dstp, hs, zf, R, H, False, cpt, 4)
    h_enc = _tc_post_enc(accf[0], accf[1], hs, p0, p1, b_enc.reshape(1, H))

    q0 = p0.reshape(Rc, 2)
    q1 = p1.reshape(Rc, 2)
    hds = _tc_prep_dec(h_enc.reshape(Rc, 2 * H), W_dec, q0, q1)
    accc = _sc_edge_pass2(srcp, dstp, hds, zc, Rc, D, True, cpt, 2)
    outd = _tc_final(accc[0], accc[1], hds, q0, q1, b_dec.reshape(1, D))

    return outd[:Nc].reshape(N, D)


# speedup vs baseline: 3.6079x; 1.1327x over previous
import functools

import jax
import jax.numpy as jnp
from jax import lax
from jax.experimental import pallas as pl
from jax.experimental.pallas import tpu as pltpu
from jax.experimental.pallas import tpu_sc as plsc

NC = 2
NS = 16
NW = NC * NS
CH = 128

_SC_PARAMS = pltpu.CompilerParams(use_tc_tiling_on_sc=False)


def _sc_degree(ep, zeros8, ones8, R, iters):
    rpt = R // NS
    mesh = plsc.VectorSubcoreMesh(core_axis_name="c", subcore_axis_name="s")
    K = 8
    rounds = iters // K

    @functools.partial(
        pl.kernel,
        out_type=jax.ShapeDtypeStruct((NC, R, 8), jnp.float32),
        mesh=mesh,
        scratch_types=[
            pltpu.VMEM((iters, CH), jnp.int32),
            pltpu.VMEM((K, CH), jnp.int32),
            pltpu.VMEM((CH, 8), jnp.float32),
            pltpu.VMEM((rpt, 8), jnp.float32),
            pltpu.VMEM_SHARED((R, 8), jnp.float32),
            pltpu.SemaphoreType.DMA,
        ],
        compiler_params=_SC_PARAMS,
    )
    def k(ep_hbm, zeros_hbm, ones_hbm, out_hbm, ep_v, idxd32, ones_v,
          chunk_v, hist, sem):
        cid = lax.axis_index("c")
        sid = lax.axis_index("s")
        wid = sid * NC + cid
        row = pl.ds(sid * rpt, rpt)
        pltpu.sync_copy(zeros_hbm.at[row], chunk_v)
        pltpu.sync_copy(chunk_v, hist.at[row])
        pltpu.sync_copy(ep_hbm.at[pl.ds(wid * iters, iters)], ep_v)
        pltpu.sync_copy(ones_hbm, ones_v)
        plsc.subcore_barrier()

        def body(g, carry):
            for b in range(K):
                for j in range(CH // 16):
                    v = ep_v[g * K + b, pl.ds(j * 16, 16)]
                    idxd32[b, pl.ds(j * 16, 16)] = v >> 16
                pltpu.async_copy(
                    ones_v, hist.at[idxd32.at[b]], sem, add=True)
            for b in range(K):
                pltpu.make_async_copy(
                    ones_v, hist.at[idxd32.at[0]], sem).wait()
            return carry

        lax.fori_loop(0, rounds, body, 0)
        plsc.subcore_barrier()
        pltpu.sync_copy(hist.at[row], chunk_v)
        pltpu.sync_copy(chunk_v, out_hbm.at[cid, row])

    return k(ep, zeros8, ones8)


def _sc_edge_pass(ep, table, zeros, R, W, shift, cpt, nb):
    rpt = R // NS
    mesh = plsc.VectorSubcoreMesh(core_axis_name="c", subcore_axis_name="s")
    NB = nb

    @functools.partial(
        pl.kernel,
        out_type=jax.ShapeDtypeStruct((NC, R, W), jnp.float32),
        mesh=mesh,
        scratch_types=(
            [pltpu.VMEM((cpt, CH), jnp.int32),
             pltpu.VMEM((NB, CH), jnp.int32),
             pltpu.VMEM((NB, CH), jnp.int32)]
            + [pltpu.VMEM((CH, W), jnp.float32) for _ in range(NB)]
            + [pltpu.VMEM_SHARED((R, W), jnp.float32),
               pltpu.VMEM_SHARED((R, W), jnp.float32)]
            + [pltpu.SemaphoreType.DMA for _ in range(2 * NB)]
        ),
        compiler_params=_SC_PARAMS,
    )
    def k(ep_hbm, table_hbm, zeros_hbm, out_hbm,
          ep_v, idxs32, idxd32, *bufs_and_sems):
        rows = bufs_and_sems[:NB]
        acc = bufs_and_sems[NB]
        table_sh = bufs_and_sems[NB + 1]
        semg = bufs_and_sems[NB + 2:NB + 2 + NB]
        sems = bufs_and_sems[NB + 2 + NB:]
        cid = lax.axis_index("c")
        sid = lax.axis_index("s")
        wid = sid * NC + cid
        row = pl.ds(sid * rpt, rpt)

        chunks = []
        o = 0
        while o < rpt:
            c = min(CH, rpt - o)
            chunks.append((o, c))
            o += c
        pltpu.sync_copy(table_hbm.at[row], table_sh.at[row])
        pltpu.sync_copy(zeros_hbm, rows[0])
        for (o, c) in chunks:
            pltpu.sync_copy(rows[0].at[pl.ds(0, c)],
                            acc.at[pl.ds(sid * rpt + o, c)])
        pltpu.sync_copy(ep_hbm.at[pl.ds(wid * cpt, cpt)], ep_v)
        plsc.subcore_barrier()

        def widen(b, i):
            for j in range(CH // 16):
                v = ep_v[i, pl.ds(j * 16, 16)]
                lo = v & 0xFFFF
                hi = v >> 16
                if shift:
                    lo = lo >> 1
                    hi = hi >> 1
                idxs32[b, pl.ds(j * 16, 16)] = lo
                idxd32[b, pl.ds(j * 16, 16)] = hi

        def body(g, carry):
            for b in range(NB):
                @pl.when(g > 0)
                def _drain():
                    pltpu.make_async_copy(
                        rows[b], acc.at[idxd32.at[0]], sems[b]).wait()
                widen(b, g * NB + b)
                pltpu.async_copy(
                    table_sh.at[idxs32.at[b]], rows[b], semg[b])
            for b in range(NB):
                pltpu.make_async_copy(
                    table_sh.at[idxs32.at[0]], rows[b], semg[b]).wait()
                pltpu.async_copy(
                    rows[b], acc.at[idxd32.at[b]], sems[b], add=True)
            return carry

        lax.fori_loop(0, cpt // NB, body, 0)
        for b in range(NB):
            pltpu.make_async_copy(
                rows[b], acc.at[idxd32.at[0]], sems[b]).wait()
        plsc.subcore_barrier()
        live = {}
        for z, (o, c) in enumerate(chunks):
            sl = pl.ds(sid * rpt + o, c)
            b = z % NB
            if b in live:
                pltpu.make_async_copy(
                    rows[b].at[pl.ds(0, live[b])],
                    out_hbm.at[cid, pl.ds(0, live[b])], semg[b]).wait()
            pltpu.sync_copy(acc.at[sl], rows[b].at[pl.ds(0, c)])
            pltpu.async_copy(rows[b].at[pl.ds(0, c)],
                             out_hbm.at[cid, sl], semg[b])
            live[b] = c
        for b, c in live.items():
            pltpu.make_async_copy(
                rows[b].at[pl.ds(0, c)],
                out_hbm.at[cid, pl.ds(0, c)], semg[b]).wait()

    return k(ep, table, zeros)


def _tc_prep_enc(x_pad, W, degp, B=640):
    R, D = x_pad.shape
    H = W.shape[1]

    def body(x_ref, w_ref, d0_ref, d1_ref, o_ref):
        dinv = lax.rsqrt(d0_ref[0, :, 0:1] + d1_ref[0, :, 0:1] + 1.0)
        o_ref[...] = jnp.dot(x_ref[...], w_ref[...],
                             preferred_element_type=jnp.float32) * dinv

    return pl.pallas_call(
        body,
        grid=(R // B,),
        in_specs=[
            pl.BlockSpec((B, D), lambda i: (i, 0)),
            pl.BlockSpec((D, H), lambda i: (0, 0)),
            pl.BlockSpec((1, B, 8), lambda i: (0, i, 0)),
            pl.BlockSpec((1, B, 8), lambda i: (1, i, 0)),
        ],
        out_specs=pl.BlockSpec((B, H), lambda i: (i, 0)),
        out_shape=jax.ShapeDtypeStruct((R, H), jnp.float32),
    )(x_pad, W, degp, degp)


def _tc_mid(a0v, a1v, hsv, degv, W, b, B=640):
    Rc, H2 = hsv.shape
    H = H2 // 2
    D = W.shape[1]

    def body(a0_ref, a1_ref, hs_ref, d0_ref, d1_ref, w_ref, b_ref, o_ref):
        dl = d0_ref[0, :, 0:1] + d1_ref[0, :, 0:1] + 1.0
        dr = d0_ref[0, :, 8:9] + d1_ref[0, :, 8:9] + 1.0
        sl_ = (a0_ref[0, :, :H] + a1_ref[0, :, :H] + hs_ref[:, :H])
        sr_ = (a0_ref[0, :, H:] + a1_ref[0, :, H:] + hs_ref[:, H:])
        hl = jnp.maximum(sl_ * lax.rsqrt(dl) + b_ref[...], 0.0)
        hr = jnp.maximum(sr_ * lax.rsqrt(dr) + b_ref[...], 0.0)
        xc = 0.5 * (hl + hr)
        degc = dl + dr - 1.0
        o_ref[...] = jnp.dot(xc, w_ref[...],
                             preferred_element_type=jnp.float32) * lax.rsqrt(degc)

    return pl.pallas_call(
        body,
        grid=(Rc // B,),
        in_specs=[
            pl.BlockSpec((1, B, H2), lambda i: (0, i, 0)),
            pl.BlockSpec((1, B, H2), lambda i: (1, i, 0)),
            pl.BlockSpec((B, H2), lambda i: (i, 0)),
            pl.BlockSpec((1, B, 16), lambda i: (0, i, 0)),
            pl.BlockSpec((1, B, 16), lambda i: (1, i, 0)),
            pl.BlockSpec((H, D), lambda i: (0, 0)),
            pl.BlockSpec((1, H), lambda i: (0, 0)),
        ],
        out_specs=pl.BlockSpec((B, D), lambda i: (i, 0)),
        out_shape=jax.ShapeDtypeStruct((Rc, D), jnp.float32),
    )(a0v, a1v, hsv, degv, degv, W, b)


def _tc_final(accc, hds, degv, b, B=640):
    Rc, D = hds.shape

    def body(a0_ref, a1_ref, hds_ref, d0_ref, d1_ref, b_ref, o_ref):
        dl = d0_ref[0, :, 0:1] + d1_ref[0, :, 0:1] + 1.0
        dr = d0_ref[0, :, 8:9] + d1_ref[0, :, 8:9] + 1.0
        degc = dl + dr - 1.0
        xd = ((a0_ref[0] + a1_ref[0] + hds_ref[...]) * lax.rsqrt(degc)
              + b_ref[...])
        o_ref[:, :D] = xd
        o_ref[:, D:] = xd

    return pl.pallas_call(
        body,
        grid=(Rc // B,),
        in_specs=[
            pl.BlockSpec((1, B, D), lambda i: (0, i, 0)),
            pl.BlockSpec((1, B, D), lambda i: (1, i, 0)),
            pl.BlockSpec((B, D), lambda i: (i, 0)),
            pl.BlockSpec((1, B, 16), lambda i: (0, i, 0)),
            pl.BlockSpec((1, B, 16), lambda i: (1, i, 0)),
            pl.BlockSpec((1, D), lambda i: (0, 0)),
        ],
        out_specs=pl.BlockSpec((B, 2 * D), lambda i: (i, 0)),
        out_shape=jax.ShapeDtypeStruct((Rc, 2 * D), jnp.float32),
    )(accc, accc, hds, degv, degv, b)


def kernel(x, edge_index, batch, W_enc, b_enc, W_dec, b_dec):
    N, D = x.shape
    H = W_enc.shape[1]
    E = edge_index.shape[1]
    Nc = N // 2

    Rc = ((Nc + 1 + 255) // 256) * 256
    R = 2 * Rc
    S = -(-(-(-E // CH)) // (NS * 8)) * 8
    cpt = -(-(-(-S // NC)) // 4) * 4
    iters = NS * S // NW
    C_pad = max(NS * S, NW * cpt)
    pad_e = C_pad * CH - E

    epk = edge_index[0] | (edge_index[1] << 16)
    ep = jnp.concatenate(
        [epk, jnp.full((pad_e,), N | (N << 16), jnp.int32)]).reshape(-1, CH)

    zeros8 = jnp.zeros((R, 8), jnp.float32)
    ones8 = jnp.zeros((CH, 8), jnp.float32).at[:, 0].set(1.0)
    zf = jnp.zeros((CH, H), jnp.float32)
    zc = jnp.zeros((CH, D), jnp.float32)
    x_pad = jnp.concatenate([x, jnp.zeros((R - N, D), x.dtype)])

    degp = _sc_degree(ep, zeros8, ones8, R, iters)
    hs = _tc_prep_enc(x_pad, W_enc, degp)
    accf = _sc_edge_pass(ep, hs, zf, R, H, False, cpt, 4)

    degv = degp.reshape(NC, Rc, 16)
    hds = _tc_mid(accf.reshape(NC, Rc, 2 * H), accf.reshape(NC, Rc, 2 * H),
                  hs.reshape(Rc, 2 * H), degv, W_dec, b_enc.reshape(1, H))
    accc = _sc_edge_pass(ep, hds, zc, Rc, D, True, cpt, 2)
    outd = _tc_final(accc, hds, degv, b_dec.reshape(1, D))

    return outd[:Nc].reshape(N, D)
